# baseline (speedup 1.0000x reference)
import functools

import jax
import jax.numpy as jnp
from jax import lax
from jax.experimental import pallas as pl
from jax.experimental.pallas import tpu as pltpu
from jax.experimental.pallas import tpu_sc as plsc

D = 64
SEQ = 200
NC, NS = 2, 16
NW = NC * NS
N_STREAM = 4
S_LEN = 100
BLK_ROWS = N_STREAM * S_LEN
LANES = 16

_MESH = plsc.VectorSubcoreMesh(
    core_axis_name="c", subcore_axis_name="s", num_cores=NC, num_subcores=NS
)


def _make_emb(n_rows):
    n_blocks = n_rows // BLK_ROWS
    blocks_per_w = n_blocks // NW

    @functools.partial(
        pl.kernel,
        out_type=jax.ShapeDtypeStruct((n_rows, D), jnp.float32),
        mesh=_MESH,
        scratch_types=[
            pltpu.VMEM((N_STREAM, S_LEN), jnp.int32),
            pltpu.VMEM((BLK_ROWS, D), jnp.float32),
            pltpu.VMEM((SEQ, D), jnp.float32),
            pltpu.SemaphoreType.DMA,
        ],
        compiler_params=pltpu.CompilerParams(use_tc_tiling_on_sc=False),
    )
    def emb(x_hbm, tok_hbm, pos_hbm, out_hbm, idx_v, rows_v, pos_v, sem):
        wid = lax.axis_index("s") * NC + lax.axis_index("c")
        base_blk = wid * blocks_per_w
        pltpu.sync_copy(pos_hbm, pos_v)

        @pl.loop(0, blocks_per_w)
        def _(i):
            blk = base_blk + i
            pltpu.sync_copy(x_hbm.at[blk], idx_v)
            cps = [
                pltpu.async_copy(
                    tok_hbm.at[idx_v.at[j]],
                    rows_v.at[pl.ds(j * S_LEN, S_LEN)],
                    sem,
                )
                for j in range(N_STREAM)
            ]
            for cp in cps:
                cp.wait()

            @pl.loop(0, SEQ)
            def _(s):
                for c in range(BLK_ROWS // SEQ):
                    r = c * SEQ + s
                    for d in range(D // LANES):
                        sl = pl.ds(d * LANES, LANES)
                        rows_v[r, sl] += pos_v[s, sl]

            pltpu.sync_copy(rows_v, out_hbm.at[pl.ds(blk * BLK_ROWS, BLK_ROWS)])

    return emb


def kernel(x, tok_emb, pos_emb):
    batch, seq = x.shape
    n_rows = batch * seq
    xb = x.reshape(n_rows // BLK_ROWS, N_STREAM, S_LEN).astype(jnp.int32)
    out = _make_emb(n_rows)(xb, tok_emb, pos_emb)
    return out.reshape(batch, seq, D)


# speedup vs baseline: 1.1215x; 1.1215x over previous
import functools

import jax
import jax.numpy as jnp
from jax import lax
from jax.experimental import pallas as pl
from jax.experimental.pallas import tpu as pltpu
from jax.experimental.pallas import tpu_sc as plsc

D = 64
SEQ = 200
NC, NS = 2, 16
NW = NC * NS
N_STREAM = 4
S_LEN = 100
BLK_ROWS = N_STREAM * S_LEN
LANES = 16

_MESH = plsc.VectorSubcoreMesh(
    core_axis_name="c", subcore_axis_name="s", num_cores=NC, num_subcores=NS
)


def _make_emb(n_rows):
    n_blocks = n_rows // BLK_ROWS
    b_w = n_blocks // NW

    @functools.partial(
        pl.kernel,
        out_type=jax.ShapeDtypeStruct((n_rows, D), jnp.float32),
        mesh=_MESH,
        scratch_types=[
            pltpu.VMEM((2, N_STREAM, S_LEN), jnp.int32),
            pltpu.VMEM((2, BLK_ROWS, D), jnp.float32),
            pltpu.VMEM((SEQ, D), jnp.float32),
            pltpu.SemaphoreType.DMA,
            pltpu.SemaphoreType.DMA,
            pltpu.SemaphoreType.DMA,
            pltpu.SemaphoreType.DMA,
            pltpu.SemaphoreType.DMA,
            pltpu.SemaphoreType.DMA,
        ],
        compiler_params=pltpu.CompilerParams(use_tc_tiling_on_sc=False),
    )
    def emb(x_hbm, tok_hbm, pos_hbm, out_hbm, idxs, rows, pos_v, g0, g1, w0, w1, s0, s1):
        gs, ws, iss = [g0, g1], [w0, w1], [s0, s1]
        wid = lax.axis_index("s") * NC + lax.axis_index("c")
        base = wid * b_w
        pltpu.sync_copy(pos_hbm, pos_v)

        def fire_gathers(b):
            for j in range(N_STREAM):
                pltpu.async_copy(
                    tok_hbm.at[idxs.at[b].at[j]],
                    rows.at[b].at[pl.ds(j * S_LEN, S_LEN)],
                    gs[b],
                )

        def wait_gather(b):
            pltpu.make_async_copy(
                tok_hbm.at[pl.ds(0, BLK_ROWS)], rows.at[b], gs[b]
            ).wait()

        def wait_wb(b):
            pltpu.make_async_copy(
                rows.at[b], out_hbm.at[pl.ds(0, BLK_ROWS)], ws[b]
            ).wait()

        pltpu.sync_copy(x_hbm.at[base], idxs.at[0])
        fire_gathers(0)
        pltpu.async_copy(x_hbm.at[base + 1], idxs.at[1], iss[1])

        @pl.loop(0, b_w, step=2)
        def _(g):
            for b in range(2):
                nb = 1 - b
                i = g + b
                blk = base + i

                @pl.when(i + 1 < b_w)
                def _():
                    pltpu.make_async_copy(
                        x_hbm.at[blk], idxs.at[nb], iss[nb]
                    ).wait()

                    @pl.when(i >= 1)
                    def _():
                        wait_wb(nb)

                    fire_gathers(nb)

                wait_gather(b)

                @pl.when(i + 2 < b_w)
                def _():
                    pltpu.async_copy(x_hbm.at[blk + 2], idxs.at[b], iss[b])

                @pl.loop(0, SEQ)
                def _(s):
                    for c in range(BLK_ROWS // SEQ):
                        r = c * SEQ + s
                        for d in range(D // LANES):
                            sl = pl.ds(d * LANES, LANES)
                            rows[b, r, sl] += pos_v[s, sl]

                pltpu.async_copy(
                    rows.at[b], out_hbm.at[pl.ds(blk * BLK_ROWS, BLK_ROWS)], ws[b]
                )

        wait_wb(0)
        wait_wb(1)

    return emb


def kernel(x, tok_emb, pos_emb):
    batch, seq = x.shape
    n_rows = batch * seq
    xb = x.reshape(n_rows // BLK_ROWS, N_STREAM, S_LEN).astype(jnp.int32)
    out = _make_emb(n_rows)(xb, tok_emb, pos_emb)
    return out.reshape(batch, seq, D)
